# baseline (device time: 15779 ns/iter reference)
import jax
import jax.numpy as jnp
from jax import lax
from jax.experimental import pallas as pl
from jax.experimental.pallas import tpu as pltpu

N_DEV = 32


def kernel(x):
    m, n = x.shape

    def body(x_ref, out_ref, v_ref, comm_ref, send_sems, recv_sems):
        my = lax.axis_index("i")

        barrier = pltpu.get_barrier_semaphore()
        for j in range(1, N_DEV):
            src = lax.rem(my - j + N_DEV, N_DEV)
            pl.semaphore_signal(
                barrier, inc=1,
                device_id=(src,), device_id_type=pl.DeviceIdType.MESH,
            )
        pl.semaphore_wait(barrier, N_DEV - 1)

        lx = jnp.log(x_ref[...].astype(jnp.float32))

        t = lx
        size = m
        while size > 1:
            half = size // 2
            t = t[:half] + t[half:size]
            size = half
        v_ref[...] = t

        rdmas = []
        for j in range(1, N_DEV):
            dst = lax.rem(my + j, N_DEV)
            rdma = pltpu.make_async_remote_copy(
                src_ref=v_ref,
                dst_ref=comm_ref.at[j],
                send_sem=send_sems.at[j],
                recv_sem=recv_sems.at[j],
                device_id=(dst,),
                device_id_type=pl.DeviceIdType.MESH,
            )
            rdma.start()
            rdmas.append(rdma)

        row = lax.broadcasted_iota(jnp.int32, (m, m), 0)
        col = lax.broadcasted_iota(jnp.int32, (m, m), 1)
        ltri = (row >= col).astype(jnp.bfloat16)
        cs = lax.dot_general(
            ltri,
            lx.astype(jnp.bfloat16),
            (((1,), (0,)), ((), ())),
            preferred_element_type=jnp.float32,
        )

        for rdma in rdmas:
            rdma.wait()

        vals = comm_ref[:, 0, :]
        srow = lax.broadcasted_iota(jnp.int32, (N_DEV, n), 0)
        srcidx = lax.rem(my - srow + N_DEV, N_DEV)
        masked = jnp.where(srcidx < my, vals, jnp.zeros_like(vals))
        size = N_DEV
        while size > 1:
            half = size // 2
            masked = masked[:half] + masked[half:size]
            size = half

        out_ref[...] = jnp.exp(cs + masked)

    return pl.pallas_call(
        body,
        out_shape=jax.ShapeDtypeStruct((m, n), jnp.float32),
        in_specs=[pl.BlockSpec(memory_space=pltpu.VMEM)],
        out_specs=pl.BlockSpec(memory_space=pltpu.VMEM),
        scratch_shapes=[
            pltpu.VMEM((1, n), jnp.float32),
            pltpu.VMEM((N_DEV, 1, n), jnp.float32),
            pltpu.SemaphoreType.DMA((N_DEV,)),
            pltpu.SemaphoreType.DMA((N_DEV,)),
        ],
        compiler_params=pltpu.CompilerParams(collective_id=0),
    )(x)


# device time: 14298 ns/iter; 1.1036x vs baseline; 1.1036x over previous
import jax
import jax.numpy as jnp
from jax import lax
from jax.experimental import pallas as pl
from jax.experimental.pallas import tpu as pltpu

N_DEV = 32


def kernel(x):
    m, n = x.shape

    def body(x_ref, out_ref, acc_ref, v_ref, comm_ref, send_sems, recv_sems):
        my = lax.axis_index("i")

        barrier = pltpu.get_barrier_semaphore()
        for j in range(1, N_DEV):
            src = lax.rem(my - j + N_DEV, N_DEV)
            pl.semaphore_signal(
                barrier, inc=1,
                device_id=(src,), device_id_type=pl.DeviceIdType.MESH,
            )
        pl.semaphore_wait(barrier, N_DEV - 1)

        xf = x_ref[...].astype(jnp.float32)
        acc_ref[...] = x_ref[...].astype(jnp.bfloat16)
        t = xf
        size = m
        while size > 1:
            half = size // 2
            t = t[:half] * t[half:size]
            size = half
        v_ref[...] = t

        rdmas = []
        for j in range(1, N_DEV):
            dst = lax.rem(my + j, N_DEV)
            rdma = pltpu.make_async_remote_copy(
                src_ref=v_ref,
                dst_ref=comm_ref.at[j],
                send_sem=send_sems.at[j],
                recv_sem=recv_sems.at[j],
                device_id=(dst,),
                device_id_type=pl.DeviceIdType.MESH,
            )
            rdma.start()
            rdmas.append(rdma)

        s = 1
        while s < m:
            prev = acc_ref[pl.ds(0, m - s), :]
            cur = acc_ref[pl.ds(s, m - s), :]
            acc_ref[pl.ds(s, m - s), :] = cur * prev
            s *= 2

        for rdma in rdmas:
            rdma.wait()

        vals = comm_ref[:, 0, :]
        row = lax.broadcasted_iota(jnp.int32, (N_DEV, n), 0)
        srcidx = lax.rem(my - row + N_DEV, N_DEV)
        masked = jnp.where(srcidx < my, vals, jnp.ones_like(vals))
        size = N_DEV
        while size > 1:
            half = size // 2
            masked = masked[:half] * masked[half:size]
            size = half
        prefix = masked

        out_ref[...] = acc_ref[...].astype(jnp.float32) * prefix

    return pl.pallas_call(
        body,
        out_shape=jax.ShapeDtypeStruct((m, n), jnp.float32),
        in_specs=[pl.BlockSpec(memory_space=pltpu.VMEM)],
        out_specs=pl.BlockSpec(memory_space=pltpu.VMEM),
        scratch_shapes=[
            pltpu.VMEM((m, n), jnp.bfloat16),
            pltpu.VMEM((1, n), jnp.float32),
            pltpu.VMEM((N_DEV, 1, n), jnp.float32),
            pltpu.SemaphoreType.DMA((N_DEV,)),
            pltpu.SemaphoreType.DMA((N_DEV,)),
        ],
        compiler_params=pltpu.CompilerParams(collective_id=0),
    )(x)


# device time: 14177 ns/iter; 1.1130x vs baseline; 1.0085x over previous
import jax
import jax.numpy as jnp
from jax import lax
from jax.experimental import pallas as pl
from jax.experimental.pallas import tpu as pltpu

N_DEV = 32
G = 8
N_GROUPS = N_DEV // G


def kernel(x):
    m, n = x.shape

    def body(
        x_ref, out_ref, acc_ref, v_ref, vg_ref,
        commA_ref, commB_ref, sendA, recvA, sendB, recvB,
    ):
        my = lax.axis_index("i")
        off = lax.rem(my, G)
        g_base = my - off
        g_id = my // G

        barrier = pltpu.get_barrier_semaphore()
        for d in range(1, G):
            src = g_base + lax.rem(off - d + G, G)
            pl.semaphore_signal(
                barrier, inc=1,
                device_id=(src,), device_id_type=pl.DeviceIdType.MESH,
            )
        for e in range(1, N_GROUPS):
            src = lax.rem(my - e * G + N_DEV, N_DEV)
            pl.semaphore_signal(
                barrier, inc=1,
                device_id=(src,), device_id_type=pl.DeviceIdType.MESH,
            )
        pl.semaphore_wait(barrier, (G - 1) + (N_GROUPS - 1))

        xf = x_ref[...].astype(jnp.float32)
        acc_ref[...] = xf
        t = xf
        size = m
        while size > 1:
            half = size // 2
            t = t[:half] * t[half:size]
            size = half
        v_ref[...] = t
        commA_ref[0, :, :] = t

        rdmasA = []
        for d in range(1, G):
            dst = g_base + lax.rem(off + d, G)
            rdma = pltpu.make_async_remote_copy(
                src_ref=v_ref,
                dst_ref=commA_ref.at[d],
                send_sem=sendA.at[d],
                recv_sem=recvA.at[d],
                device_id=(dst,),
                device_id_type=pl.DeviceIdType.MESH,
            )
            rdma.start()
            rdmasA.append(rdma)

        s = 1
        while s < 32:
            prev = acc_ref[pl.ds(0, m - s), :]
            cur = acc_ref[pl.ds(s, m - s), :]
            acc_ref[pl.ds(s, m - s), :] = cur * prev
            s *= 2

        for rdma in rdmasA:
            rdma.wait()

        valsA = commA_ref[:, 0, :]
        rowA = lax.broadcasted_iota(jnp.int32, (G, n), 0)
        srcoff = lax.rem(off - rowA + G, G)
        maskedA = jnp.where(srcoff < off, valsA, jnp.ones_like(valsA))
        gt = valsA
        intra = maskedA
        size = G
        while size > 1:
            half = size // 2
            gt = gt[:half] * gt[half:size]
            intra = intra[:half] * intra[half:size]
            size = half
        vg_ref[...] = gt

        rdmasB = []
        for e in range(1, N_GROUPS):
            dst = lax.rem(my + e * G, N_DEV)
            rdma = pltpu.make_async_remote_copy(
                src_ref=vg_ref,
                dst_ref=commB_ref.at[e],
                send_sem=sendB.at[e],
                recv_sem=recvB.at[e],
                device_id=(dst,),
                device_id_type=pl.DeviceIdType.MESH,
            )
            rdma.start()
            rdmasB.append(rdma)

        while s < m:
            prev = acc_ref[pl.ds(0, m - s), :]
            cur = acc_ref[pl.ds(s, m - s), :]
            acc_ref[pl.ds(s, m - s), :] = cur * prev
            s *= 2

        for rdma in rdmasB:
            rdma.wait()

        valsB = commB_ref[:, 0, :]
        rowB = lax.broadcasted_iota(jnp.int32, (N_GROUPS, n), 0)
        srcg = lax.rem(g_id - rowB + N_GROUPS, N_GROUPS)
        maskedB = jnp.where(srcg < g_id, valsB, jnp.ones_like(valsB))
        size = N_GROUPS
        while size > 1:
            half = size // 2
            maskedB = maskedB[:half] * maskedB[half:size]
            size = half

        prefix = maskedB * intra
        out_ref[...] = acc_ref[...] * prefix

    return pl.pallas_call(
        body,
        out_shape=jax.ShapeDtypeStruct((m, n), jnp.float32),
        in_specs=[pl.BlockSpec(memory_space=pltpu.VMEM)],
        out_specs=pl.BlockSpec(memory_space=pltpu.VMEM),
        scratch_shapes=[
            pltpu.VMEM((m, n), jnp.float32),
            pltpu.VMEM((1, n), jnp.float32),
            pltpu.VMEM((1, n), jnp.float32),
            pltpu.VMEM((G, 1, n), jnp.float32),
            pltpu.VMEM((N_GROUPS, 1, n), jnp.float32),
            pltpu.SemaphoreType.DMA((G,)),
            pltpu.SemaphoreType.DMA((G,)),
            pltpu.SemaphoreType.DMA((N_GROUPS,)),
            pltpu.SemaphoreType.DMA((N_GROUPS,)),
        ],
        compiler_params=pltpu.CompilerParams(collective_id=0),
    )(x)
